# baseline (speedup 1.0000x reference)
import functools

import jax
import jax.numpy as jnp
from jax.experimental import pallas as pl
from jax.experimental.pallas import tpu as pltpu

_TILE = 512
_KERNEL_NUM = 5
_LOG2E = 1.4426950408889634


def _prep_kernel(src_ref, tgt_ref, tot_ref, rr_ref, scal_ref, *, n, b):
    s = src_ref[...]
    t = tgt_ref[...]
    mean = (jnp.sum(s, axis=0, keepdims=True)
            + jnp.sum(t, axis=0, keepdims=True)) * (1.0 / n)
    sb = (s - mean).astype(jnp.bfloat16)
    tb = (t - mean).astype(jnp.bfloat16)
    tot_ref[:b] = sb
    tot_ref[b:] = tb
    sf = sb.astype(jnp.float32)
    tf = tb.astype(jnp.float32)
    sq_s = jnp.sum(sf * sf, axis=1, keepdims=True)
    sq_t = jnp.sum(tf * tf, axis=1, keepdims=True)
    col = jnp.sum(sf, axis=0, keepdims=True) + jnp.sum(tf, axis=0, keepdims=True)
    ssq = jnp.sum(sq_s) + jnp.sum(sq_t)
    sum_l2 = 2.0 * n * ssq - 2.0 * jnp.sum(col * col)
    bandwidth = sum_l2 / float(n * n - n)
    c4 = -_LOG2E / (4.0 * bandwidth)
    rr_ref[:b] = sq_s * c4
    rr_ref[b:] = sq_t * c4
    scal_ref[0, 0] = -2.0 * c4


def _sym_kernel(scal_ref, tot_ref, rr_ref, rc_ref, out_ref,
                vbuf, tbuf, vsem, tsem, *, tiles, half, tile):
    T, H = tiles, half
    i = pl.program_id(0)
    dj = pl.program_id(1)
    j = jax.lax.rem(i + dj, T)
    step = i * (H + 1) + dj
    slot = jax.lax.rem(step, 2)

    dst0 = out_ref.at[pl.ds(0, tile), pl.ds(0, tile)]

    @pl.when(step >= 2)
    def _():
        pltpu.make_async_copy(vbuf.at[slot], dst0, vsem.at[slot]).wait()

    @pl.when((step >= 2) & (dj != 2))
    def _():
        pltpu.make_async_copy(tbuf.at[slot], dst0, tsem.at[slot]).wait()

    m2 = scal_ref[0, 0]
    xr = tot_ref[pl.ds(i * tile, tile), :]
    xc = tot_ref[pl.ds(j * tile, tile), :]
    gram = jax.lax.dot_general(
        xr, xc, (((1,), (1,)), ((), ())),
        preferred_element_type=jnp.float32)
    t = gram * m2 + (rr_ref[pl.ds(i * tile, tile), :]
                     + rc_ref[:, pl.ds(j * tile, tile)])
    e = jnp.exp2(t)
    acc = e
    for _ in range(_KERNEL_NUM - 1):
        e = e * e
        acc = acc + e

    vbuf[slot] = acc
    pltpu.make_async_copy(
        vbuf.at[slot],
        out_ref.at[pl.ds(i * tile, tile), pl.ds(j * tile, tile)],
        vsem.at[slot]).start()

    @pl.when(dj > 0)
    def _():
        tbuf[slot] = acc.T
        pltpu.make_async_copy(
            tbuf.at[slot],
            out_ref.at[pl.ds(j * tile, tile), pl.ds(i * tile, tile)],
            tsem.at[slot]).start()

    @pl.when(step == T * (H + 1) - 1)
    def _():
        for s in (0, 1):
            pltpu.make_async_copy(vbuf.at[s], dst0, vsem.at[s]).wait()
            pltpu.make_async_copy(tbuf.at[s], dst0, tsem.at[s]).wait()


def kernel(source, target):
    b, d = int(source.shape[0]), int(source.shape[1])
    n = b + int(target.shape[0])

    tot_bf, rr, scal = pl.pallas_call(
        functools.partial(_prep_kernel, n=n, b=b),
        out_shape=(
            jax.ShapeDtypeStruct((n, d), jnp.bfloat16),
            jax.ShapeDtypeStruct((n, 1), jnp.float32),
            jax.ShapeDtypeStruct((1, 1), jnp.float32),
        ),
        out_specs=(
            pl.BlockSpec(memory_space=pltpu.MemorySpace.VMEM),
            pl.BlockSpec(memory_space=pltpu.MemorySpace.VMEM),
            pl.BlockSpec(memory_space=pltpu.MemorySpace.SMEM),
        ),
        compiler_params=pltpu.CompilerParams(
            vmem_limit_bytes=100 * 1024 * 1024),
    )(source, target)
    rc = rr.reshape(1, n)

    tiles = n // _TILE
    half = tiles // 2
    grid = (tiles, half + 1)
    body = functools.partial(_sym_kernel, tiles=tiles, half=half, tile=_TILE)
    cost = pl.CostEstimate(
        flops=int(2 * n * n * d // 2 + 11 * n * n // 2),
        transcendentals=int(n * n // 2),
        bytes_accessed=int(2 * n * d * 2 + n * n * 4))
    out = pl.pallas_call(
        body,
        out_shape=jax.ShapeDtypeStruct((n, n), jnp.float32),
        grid=grid,
        in_specs=[
            pl.BlockSpec(memory_space=pltpu.MemorySpace.SMEM),
            pl.BlockSpec((n, d), lambda i, dj: (0, 0)),
            pl.BlockSpec((n, 1), lambda i, dj: (0, 0)),
            pl.BlockSpec((1, n), lambda i, dj: (0, 0)),
        ],
        out_specs=pl.BlockSpec(memory_space=pltpu.MemorySpace.HBM),
        scratch_shapes=[
            pltpu.VMEM((2, _TILE, _TILE), jnp.float32),
            pltpu.VMEM((2, _TILE, _TILE), jnp.float32),
            pltpu.SemaphoreType.DMA((2,)),
            pltpu.SemaphoreType.DMA((2,)),
        ],
        compiler_params=pltpu.CompilerParams(
            dimension_semantics=("arbitrary", "arbitrary"),
            vmem_limit_bytes=100 * 1024 * 1024),
        cost_estimate=cost,
    )(scal, tot_bf, rr, rc)
    return out
